# baseline (speedup 1.0000x reference)
import functools

import jax
import jax.numpy as jnp
from jax import lax
from jax.experimental import pallas as pl
from jax.experimental.pallas import tpu as pltpu
from jax.experimental.pallas import tpu_sc as plsc

T = 4096
D = 1024
F = 1408
E = 8
BT = 256
N_TB = T // BT

B = 256
B_LOG2 = 8
GMAX = (T * 2) // B + E
S = GMAX * B

NC = 2
NS = 16
NW = NC * NS
TPW = T // NW
SPW = S // NW
GCH = 40
CCH = 16


def _col2row(col, n):
    eye = (lax.broadcasted_iota(jnp.int32, (n, n), 0)
           == lax.broadcasted_iota(jnp.int32, (n, n), 1))
    return jnp.sum(jnp.where(eye, col, jnp.zeros_like(col)), axis=0)


def _router_body(x_ref, wg_ref, idx0_ref, idx1_ref, rank0_ref, rank1_ref,
                 w0_ref, w1_ref, base_ref, nb_ref, cumnb_ref, carry_ref):
    tb = pl.program_id(0)
    xb = x_ref[...]
    logits = jnp.dot(xb, wg_ref[...], preferred_element_type=jnp.float32)
    m = jnp.max(logits, axis=1, keepdims=True)
    ex = jnp.exp(logits - m)
    p = ex / jnp.sum(ex, axis=1, keepdims=True)
    iota = lax.broadcasted_iota(jnp.int32, (BT, E), 1)
    m1 = jnp.max(p, axis=1, keepdims=True)
    i1 = jnp.min(jnp.where(p >= m1, iota, E), axis=1, keepdims=True)
    sel1 = iota == i1
    p2 = jnp.where(sel1, -jnp.inf, p)
    m2 = jnp.max(p2, axis=1, keepdims=True)
    i2 = jnp.min(jnp.where(p2 >= m2, iota, E), axis=1, keepdims=True)
    sel2 = iota == i2
    s = m1 + m2

    @pl.when(tb == 0)
    def _():
        carry_ref[...] = jnp.zeros((1, E), jnp.float32)

    carry = carry_ref[...]
    oh = sel1.astype(jnp.float32) + sel2.astype(jnp.float32)
    tri = (lax.broadcasted_iota(jnp.int32, (BT, BT), 0)
           > lax.broadcasted_iota(jnp.int32, (BT, BT), 1)).astype(jnp.float32)
    cum_excl = carry + jnp.dot(tri, oh, preferred_element_type=jnp.float32)
    zero = jnp.zeros((BT, E), jnp.float32)
    rank0 = jnp.sum(jnp.where(sel1, cum_excl, zero), axis=1, keepdims=True)
    rank1 = jnp.sum(jnp.where(sel2, cum_excl, zero), axis=1, keepdims=True)
    new_carry = carry + jnp.sum(oh, axis=0, keepdims=True)
    carry_ref[...] = new_carry

    idx0_ref[...] = _col2row(i1, BT)
    idx1_ref[...] = _col2row(i2, BT)
    rank0_ref[...] = _col2row(rank0.astype(jnp.int32), BT)
    rank1_ref[...] = _col2row(rank1.astype(jnp.int32), BT)
    w0_ref[...] = _col2row(m1 / s, BT)
    w1_ref[...] = _col2row(m2 / s, BT)
    padded = jnp.ceil(new_carry / B) * B
    nb = padded / B
    strict = (lax.broadcasted_iota(jnp.int32, (E, E), 0)
              < lax.broadcasted_iota(jnp.int32, (E, E), 1)).astype(jnp.float32)
    base_excl = jnp.dot(padded, strict, preferred_element_type=jnp.float32)
    cumnb_excl = jnp.dot(nb, strict, preferred_element_type=jnp.float32)
    zpad = jnp.zeros((1, E), jnp.int32)

    def _pack16(row):
        return jnp.concatenate([row.astype(jnp.int32), zpad], axis=1).reshape(16)

    base_ref[...] = _pack16(base_excl)
    nb_ref[...] = _pack16(nb)
    cumnb_ref[...] = _pack16(cumnb_excl)


def _router(x, wg):
    return pl.pallas_call(
        _router_body,
        grid=(N_TB,),
        in_specs=[
            pl.BlockSpec((BT, D), lambda t: (t, 0)),
            pl.BlockSpec((D, E), lambda t: (0, 0)),
        ],
        out_specs=[
            pl.BlockSpec((BT,), lambda t: (t,)),
            pl.BlockSpec((BT,), lambda t: (t,)),
            pl.BlockSpec((BT,), lambda t: (t,)),
            pl.BlockSpec((BT,), lambda t: (t,)),
            pl.BlockSpec((BT,), lambda t: (t,)),
            pl.BlockSpec((BT,), lambda t: (t,)),
            pl.BlockSpec((16,), lambda t: (0,)),
            pl.BlockSpec((16,), lambda t: (0,)),
            pl.BlockSpec((16,), lambda t: (0,)),
        ],
        out_shape=[
            jax.ShapeDtypeStruct((T,), jnp.int32),
            jax.ShapeDtypeStruct((T,), jnp.int32),
            jax.ShapeDtypeStruct((T,), jnp.int32),
            jax.ShapeDtypeStruct((T,), jnp.int32),
            jax.ShapeDtypeStruct((T,), jnp.float32),
            jax.ShapeDtypeStruct((T,), jnp.float32),
            jax.ShapeDtypeStruct((16,), jnp.int32),
            jax.ShapeDtypeStruct((16,), jnp.int32),
            jax.ShapeDtypeStruct((16,), jnp.int32),
        ],
        scratch_shapes=[pltpu.VMEM((1, E), jnp.float32)],
    )(x, wg)


@functools.cache
def _sc_mesh():
    return plsc.VectorSubcoreMesh(core_axis_name="c", subcore_axis_name="s",
                                  num_cores=NC, num_subcores=NS)


TPC = T // NS


def _dispatch_body(base_hbm, nb_hbm, cumnb_hbm, idx0_hbm, idx1_hbm,
                   rank0_hbm, rank1_hbm, w0_hbm, w1_hbm, x_hbm,
                   wsort_hbm, pos0_hbm, pos1_hbm, be_hbm, xs_hbm,
                   base_v, nb_v, cumnb_v, idx0_v, idx1_v, rank0_v, rank1_v,
                   w0_v, w1_v, p0_v, p1_v, tok_v, slot_v, w_own_v, bev_v,
                   stok_sh, sw_sh, rows0_v, rows1_v, sem0, sem1):
    cid = lax.axis_index("c")
    sid = lax.axis_index("s")
    wid = sid * NC + cid
    tbase = sid * TPC

    cps = [pltpu.async_copy(src, dst, sem0) for src, dst in (
        (base_hbm, base_v), (nb_hbm, nb_v), (cumnb_hbm, cumnb_v),
        (idx0_hbm.at[pl.ds(tbase, TPC)], idx0_v),
        (idx1_hbm.at[pl.ds(tbase, TPC)], idx1_v),
        (rank0_hbm.at[pl.ds(tbase, TPC)], rank0_v),
        (rank1_hbm.at[pl.ds(tbase, TPC)], rank1_v),
        (w0_hbm.at[pl.ds(tbase, TPC)], w0_v),
        (w1_hbm.at[pl.ds(tbase, TPC)], w1_v))]
    for cp in cps:
        cp.wait()

    lane = lax.broadcasted_iota(jnp.int32, (16,), 0)

    def pbody(i, c):
        sl = pl.ds(i * 16, 16)
        tok_v[sl] = tbase + i * 16 + lane
        p0_v[sl] = plsc.load_gather(base_v, [idx0_v[sl]]) + rank0_v[sl]
        p1_v[sl] = plsc.load_gather(base_v, [idx1_v[sl]]) + rank1_v[sl]
        return c

    lax.fori_loop(0, TPC // 16, pbody, 0)

    pltpu.sync_copy(tok_v, stok_sh.at[p0_v])
    pltpu.sync_copy(tok_v, stok_sh.at[p1_v])
    pltpu.sync_copy(w0_v, sw_sh.at[p0_v])
    pltpu.sync_copy(w1_v, sw_sh.at[p1_v])

    @pl.when(cid == 0)
    def _():
        pltpu.sync_copy(p0_v, pos0_hbm.at[pl.ds(tbase, TPC)])
        pltpu.sync_copy(p1_v, pos1_hbm.at[pl.ds(tbase, TPC)])

    @pl.when(wid == 0)
    def _():
        nb = nb_v[...]
        cumnb_excl = cumnb_v[...]

        def ib(i, c):
            bev_v[pl.ds(i * 16, 16)] = jnp.full((16,), E - 1, jnp.int32)
            return c

        lax.fori_loop(0, 3, ib, 0)
        for j in range(16):
            plsc.store_scatter(bev_v, [cumnb_excl + j], lane, mask=j < nb)
        gu = cumnb_excl + nb
        plsc.store_scatter(bev_v, [jnp.where(lane == 7, GMAX, 47)], gu,
                           mask=lane == 7)
        pltpu.sync_copy(bev_v, be_hbm)

    plsc.subcore_barrier()

    pltpu.sync_copy(sw_sh.at[pl.ds(wid * SPW, SPW)], w_own_v)
    pltpu.sync_copy(w_own_v, wsort_hbm.at[pl.ds(wid * SPW, SPW)])
    pltpu.sync_copy(stok_sh.at[pl.ds(wid * SPW, SPW)], slot_v)

    def cl(i, c):
        sl = pl.ds(i * 16, 16)
        slot_v[sl] = slot_v[sl] & (T - 1)
        return c

    lax.fori_loop(0, SPW // 16, cl, 0)

    rows = (rows0_v, rows1_v)
    sems = (sem0, sem1)
    nch = SPW // GCH

    def issue(ci):
        return pltpu.async_copy(x_hbm.at[slot_v.at[pl.ds(ci * GCH, GCH)]],
                                rows[ci % 2], sems[ci % 2])

    inflight = issue(0)
    for ci in range(nch):
        nxt = issue(ci + 1) if ci + 1 < nch else None
        inflight.wait()
        pltpu.sync_copy(rows[ci % 2],
                        xs_hbm.at[pl.ds(wid * SPW + ci * GCH, GCH)])
        inflight = nxt


@functools.cache
def _dispatch():
    return pl.kernel(
        _dispatch_body,
        out_type=[
            jax.ShapeDtypeStruct((S,), jnp.float32),
            jax.ShapeDtypeStruct((T,), jnp.int32),
            jax.ShapeDtypeStruct((T,), jnp.int32),
            jax.ShapeDtypeStruct((48,), jnp.int32),
            jax.ShapeDtypeStruct((S, D), jnp.float32),
        ],
        mesh=_sc_mesh(),
        scratch_types=[
            pltpu.VMEM((16,), jnp.int32),
            pltpu.VMEM((16,), jnp.int32),
            pltpu.VMEM((16,), jnp.int32),
            pltpu.VMEM((TPC,), jnp.int32),
            pltpu.VMEM((TPC,), jnp.int32),
            pltpu.VMEM((TPC,), jnp.int32),
            pltpu.VMEM((TPC,), jnp.int32),
            pltpu.VMEM((TPC,), jnp.float32),
            pltpu.VMEM((TPC,), jnp.float32),
            pltpu.VMEM((TPC,), jnp.int32),
            pltpu.VMEM((TPC,), jnp.int32),
            pltpu.VMEM((TPC,), jnp.int32),
            pltpu.VMEM((SPW,), jnp.int32),
            pltpu.VMEM((SPW,), jnp.float32),
            pltpu.VMEM((48,), jnp.int32),
            pltpu.VMEM_SHARED((S,), jnp.int32),
            pltpu.VMEM_SHARED((S,), jnp.float32),
            pltpu.VMEM((GCH, D), jnp.float32),
            pltpu.VMEM((GCH, D), jnp.float32),
            pltpu.SemaphoreType.DMA,
            pltpu.SemaphoreType.DMA,
        ],
        compiler_params=pltpu.CompilerParams(needs_layout_passes=False),
    )


def _gemm_body(be_ref, xs_ref, w1_ref, w3_ref, w2_ref, ws_ref, out_ref,
               w1b_ref, w3b_ref, w2b_ref):
    g_id = pl.program_id(0)
    live = g_id < be_ref[GMAX]
    new_expert = jnp.logical_or(
        g_id == 0, be_ref[g_id] != be_ref[jnp.maximum(g_id - 1, 0)])

    @pl.when(jnp.logical_and(live, new_expert))
    def _():
        w1b_ref[...] = w1_ref[0].astype(jnp.bfloat16)
        w3b_ref[...] = w3_ref[0].astype(jnp.bfloat16)
        w2b_ref[...] = w2_ref[0].astype(jnp.bfloat16)

    @pl.when(live)
    def _():
        xb = xs_ref[...].astype(jnp.bfloat16)
        h1 = jnp.dot(xb, w1b_ref[...], preferred_element_type=jnp.float32)
        h3 = jnp.dot(xb, w3b_ref[...], preferred_element_type=jnp.float32)
        g = h1 * jax.nn.sigmoid(h1) * h3
        o = jnp.dot(g.astype(jnp.bfloat16), w2b_ref[...],
                    preferred_element_type=jnp.float32)
        wrow = ws_ref[...].reshape(1, B)
        eye = (lax.broadcasted_iota(jnp.int32, (B, B), 0)
               == lax.broadcasted_iota(jnp.int32, (B, B), 1))
        wcol = jnp.sum(jnp.where(eye, wrow, jnp.zeros((B, B), jnp.float32)),
                       axis=1, keepdims=True)
        out_ref[...] = o * wcol


def _gemm(be, xs, w1, w3, w2, ws):
    grid_spec = pltpu.PrefetchScalarGridSpec(
        num_scalar_prefetch=1,
        grid=(GMAX,),
        in_specs=[
            pl.BlockSpec((B, D), lambda g, be_ref: (g, 0)),
            pl.BlockSpec((1, D, F), lambda g, be_ref: (be_ref[g], 0, 0)),
            pl.BlockSpec((1, D, F), lambda g, be_ref: (be_ref[g], 0, 0)),
            pl.BlockSpec((1, F, D), lambda g, be_ref: (be_ref[g], 0, 0)),
            pl.BlockSpec((B,), lambda g, be_ref: (g,)),
        ],
        out_specs=pl.BlockSpec((B, D), lambda g, be_ref: (g, 0)),
        scratch_shapes=[
            pltpu.VMEM((D, F), jnp.bfloat16),
            pltpu.VMEM((D, F), jnp.bfloat16),
            pltpu.VMEM((F, D), jnp.bfloat16),
        ],
    )
    return pl.pallas_call(
        _gemm_body,
        grid_spec=grid_spec,
        out_shape=jax.ShapeDtypeStruct((S, D), jnp.float32),
        compiler_params=pltpu.CompilerParams(
            dimension_semantics=("arbitrary",),
        ),
    )(be, xs, w1, w3, w2, ws)


def _combine_body(outs_hbm, pos0_hbm, pos1_hbm, y_hbm,
                  p0_v, p1_v, a0_v, a1_v, b0_v, b1_v,
                  sa0, sa1, sb0, sb1):
    wid = lax.axis_index("s") * NC + lax.axis_index("c")
    pltpu.sync_copy(pos0_hbm.at[pl.ds(wid * TPW, TPW)], p0_v)
    pltpu.sync_copy(pos1_hbm.at[pl.ds(wid * TPW, TPW)], p1_v)

    sets = ((a0_v, a1_v, sa0, sa1), (b0_v, b1_v, sb0, sb1))
    nch = TPW // CCH

    def issue(ci):
        r0, r1, s0, s1 = sets[ci % 2]
        return (
            pltpu.async_copy(outs_hbm.at[p0_v.at[pl.ds(ci * CCH, CCH)]],
                             r0, s0),
            pltpu.async_copy(outs_hbm.at[p1_v.at[pl.ds(ci * CCH, CCH)]],
                             r1, s1),
        )

    inflight = issue(0)
    for ci in range(nch):
        nxt = issue(ci + 1) if ci + 1 < nch else None
        inflight[0].wait()
        inflight[1].wait()
        r0, r1, _, _ = sets[ci % 2]

        def arow(rr, c2, r0=r0, r1=r1):
            for cc in range(D // 16):
                sl = pl.ds(cc * 16, 16)
                r0[rr, sl] = r0[rr, sl] + r1[rr, sl]
            return c2

        lax.fori_loop(0, CCH, arow, 0)
        pltpu.sync_copy(r0, y_hbm.at[pl.ds(wid * TPW + ci * CCH, CCH)])
        inflight = nxt


@functools.cache
def _combine():
    return pl.kernel(
        _combine_body,
        out_type=jax.ShapeDtypeStruct((T, D), jnp.float32),
        mesh=_sc_mesh(),
        scratch_types=[
            pltpu.VMEM((TPW,), jnp.int32),
            pltpu.VMEM((TPW,), jnp.int32),
            pltpu.VMEM((CCH, D), jnp.float32),
            pltpu.VMEM((CCH, D), jnp.float32),
            pltpu.VMEM((CCH, D), jnp.float32),
            pltpu.VMEM((CCH, D), jnp.float32),
            pltpu.SemaphoreType.DMA,
            pltpu.SemaphoreType.DMA,
            pltpu.SemaphoreType.DMA,
            pltpu.SemaphoreType.DMA,
        ],
        compiler_params=pltpu.CompilerParams(needs_layout_passes=False),
    )


def kernel(hidden_states, W_gate, W1, W3, W2):
    x = hidden_states
    idx0, idx1, rank0, rank1, w0, w1, base16, nb16, cumnb16 = _router(x, W_gate)
    wsort, pos0, pos1, be, xs = _dispatch()(
        base16, nb16, cumnb16, idx0, idx1, rank0, rank1, w0, w1, x)
    outs = _gemm(be, xs, W1, W3, W2, wsort)
    y = _combine()(outs, pos0, pos1)
    return y


# speedup vs baseline: 1.0257x; 1.0257x over previous
import functools

import jax
import jax.numpy as jnp
from jax import lax
from jax.experimental import pallas as pl
from jax.experimental.pallas import tpu as pltpu
from jax.experimental.pallas import tpu_sc as plsc

T = 4096
D = 1024
F = 1408
E = 8
BT = 256
N_TB = T // BT

B = 256
B_LOG2 = 8
GMAX = (T * 2) // B + E
S = GMAX * B

NC = 2
NS = 16
NW = NC * NS
TPW = T // NW
SPW = S // NW
GCH = 40
CCH = 16


def _col2row(col, n):
    eye = (lax.broadcasted_iota(jnp.int32, (n, n), 0)
           == lax.broadcasted_iota(jnp.int32, (n, n), 1))
    return jnp.sum(jnp.where(eye, col, jnp.zeros_like(col)), axis=0)


def _router_body(x_ref, wg_ref, idx0_ref, idx1_ref, rank0_ref, rank1_ref,
                 w0_ref, w1_ref, base_ref, nb_ref, cumnb_ref, carry_ref):
    tb = pl.program_id(0)
    xb = x_ref[...]
    logits = jnp.dot(xb, wg_ref[...], preferred_element_type=jnp.float32)
    m = jnp.max(logits, axis=1, keepdims=True)
    ex = jnp.exp(logits - m)
    p = ex / jnp.sum(ex, axis=1, keepdims=True)
    iota = lax.broadcasted_iota(jnp.int32, (BT, E), 1)
    m1 = jnp.max(p, axis=1, keepdims=True)
    i1 = jnp.min(jnp.where(p >= m1, iota, E), axis=1, keepdims=True)
    sel1 = iota == i1
    p2 = jnp.where(sel1, -jnp.inf, p)
    m2 = jnp.max(p2, axis=1, keepdims=True)
    i2 = jnp.min(jnp.where(p2 >= m2, iota, E), axis=1, keepdims=True)
    sel2 = iota == i2
    s = m1 + m2

    @pl.when(tb == 0)
    def _():
        carry_ref[...] = jnp.zeros((1, E), jnp.float32)

    carry = carry_ref[...]
    oh = sel1.astype(jnp.float32) + sel2.astype(jnp.float32)
    tri = (lax.broadcasted_iota(jnp.int32, (BT, BT), 0)
           > lax.broadcasted_iota(jnp.int32, (BT, BT), 1)).astype(jnp.float32)
    cum_excl = carry + jnp.dot(tri, oh, preferred_element_type=jnp.float32)
    zero = jnp.zeros((BT, E), jnp.float32)
    rank0 = jnp.sum(jnp.where(sel1, cum_excl, zero), axis=1, keepdims=True)
    rank1 = jnp.sum(jnp.where(sel2, cum_excl, zero), axis=1, keepdims=True)
    new_carry = carry + jnp.sum(oh, axis=0, keepdims=True)
    carry_ref[...] = new_carry

    idx0_ref[...] = _col2row(i1, BT)
    idx1_ref[...] = _col2row(i2, BT)
    rank0_ref[...] = _col2row(rank0.astype(jnp.int32), BT)
    rank1_ref[...] = _col2row(rank1.astype(jnp.int32), BT)
    w0_ref[...] = _col2row(m1 / s, BT)
    w1_ref[...] = _col2row(m2 / s, BT)
    padded = jnp.ceil(new_carry / B) * B
    nb = padded / B
    strict = (lax.broadcasted_iota(jnp.int32, (E, E), 0)
              < lax.broadcasted_iota(jnp.int32, (E, E), 1)).astype(jnp.float32)
    base_excl = jnp.dot(padded, strict, preferred_element_type=jnp.float32)
    cumnb_excl = jnp.dot(nb, strict, preferred_element_type=jnp.float32)
    zpad = jnp.zeros((1, E), jnp.int32)

    def _pack16(row):
        return jnp.concatenate([row.astype(jnp.int32), zpad], axis=1).reshape(16)

    base_ref[...] = _pack16(base_excl)
    nb_ref[...] = _pack16(nb)
    cumnb_ref[...] = _pack16(cumnb_excl)


def _router(x, wg):
    return pl.pallas_call(
        _router_body,
        grid=(N_TB,),
        in_specs=[
            pl.BlockSpec((BT, D), lambda t: (t, 0)),
            pl.BlockSpec((D, E), lambda t: (0, 0)),
        ],
        out_specs=[
            pl.BlockSpec((BT,), lambda t: (t,)),
            pl.BlockSpec((BT,), lambda t: (t,)),
            pl.BlockSpec((BT,), lambda t: (t,)),
            pl.BlockSpec((BT,), lambda t: (t,)),
            pl.BlockSpec((BT,), lambda t: (t,)),
            pl.BlockSpec((BT,), lambda t: (t,)),
            pl.BlockSpec((16,), lambda t: (0,)),
            pl.BlockSpec((16,), lambda t: (0,)),
            pl.BlockSpec((16,), lambda t: (0,)),
        ],
        out_shape=[
            jax.ShapeDtypeStruct((T,), jnp.int32),
            jax.ShapeDtypeStruct((T,), jnp.int32),
            jax.ShapeDtypeStruct((T,), jnp.int32),
            jax.ShapeDtypeStruct((T,), jnp.int32),
            jax.ShapeDtypeStruct((T,), jnp.float32),
            jax.ShapeDtypeStruct((T,), jnp.float32),
            jax.ShapeDtypeStruct((16,), jnp.int32),
            jax.ShapeDtypeStruct((16,), jnp.int32),
            jax.ShapeDtypeStruct((16,), jnp.int32),
        ],
        scratch_shapes=[pltpu.VMEM((1, E), jnp.float32)],
    )(x, wg)


@functools.cache
def _sc_mesh():
    return plsc.VectorSubcoreMesh(core_axis_name="c", subcore_axis_name="s",
                                  num_cores=NC, num_subcores=NS)


TPC = T // NS


def _dispatch_body(base_hbm, nb_hbm, cumnb_hbm, idx0_hbm, idx1_hbm,
                   rank0_hbm, rank1_hbm, w0_hbm, w1_hbm, x_hbm,
                   wsort_hbm, pos0_hbm, pos1_hbm, be_hbm, xs_hbm,
                   base_v, nb_v, cumnb_v, idx0_v, idx1_v, rank0_v, rank1_v,
                   w0_v, w1_v, p0_v, p1_v, tok_v, slot_v, w_own_v, bev_v,
                   stok_sh, sw_sh, rows0_v, rows1_v, sem0, sem1):
    cid = lax.axis_index("c")
    sid = lax.axis_index("s")
    wid = sid * NC + cid
    tbase = sid * TPC

    cps = [pltpu.async_copy(src, dst, sem0) for src, dst in (
        (base_hbm, base_v), (nb_hbm, nb_v), (cumnb_hbm, cumnb_v),
        (idx0_hbm.at[pl.ds(tbase, TPC)], idx0_v),
        (idx1_hbm.at[pl.ds(tbase, TPC)], idx1_v),
        (rank0_hbm.at[pl.ds(tbase, TPC)], rank0_v),
        (rank1_hbm.at[pl.ds(tbase, TPC)], rank1_v),
        (w0_hbm.at[pl.ds(tbase, TPC)], w0_v),
        (w1_hbm.at[pl.ds(tbase, TPC)], w1_v))]
    for cp in cps:
        cp.wait()

    lane = lax.broadcasted_iota(jnp.int32, (16,), 0)

    def pbody(i, c):
        sl = pl.ds(i * 16, 16)
        tok_v[sl] = tbase + i * 16 + lane
        p0_v[sl] = plsc.load_gather(base_v, [idx0_v[sl]]) + rank0_v[sl]
        p1_v[sl] = plsc.load_gather(base_v, [idx1_v[sl]]) + rank1_v[sl]
        return c

    lax.fori_loop(0, TPC // 16, pbody, 0)

    pltpu.sync_copy(tok_v, stok_sh.at[p0_v])
    pltpu.sync_copy(tok_v, stok_sh.at[p1_v])
    pltpu.sync_copy(w0_v, sw_sh.at[p0_v])
    pltpu.sync_copy(w1_v, sw_sh.at[p1_v])

    @pl.when(cid == 0)
    def _():
        pltpu.sync_copy(p0_v, pos0_hbm.at[pl.ds(tbase, TPC)])
        pltpu.sync_copy(p1_v, pos1_hbm.at[pl.ds(tbase, TPC)])

    @pl.when(wid == 0)
    def _():
        nb = nb_v[...]
        cumnb_excl = cumnb_v[...]

        def ib(i, c):
            bev_v[pl.ds(i * 16, 16)] = jnp.full((16,), E - 1, jnp.int32)
            return c

        lax.fori_loop(0, 3, ib, 0)
        for j in range(16):
            plsc.store_scatter(bev_v, [cumnb_excl + j], lane, mask=j < nb)
        gu = cumnb_excl + nb
        plsc.store_scatter(bev_v, [jnp.where(lane == 7, GMAX, 47)], gu,
                           mask=lane == 7)
        pltpu.sync_copy(bev_v, be_hbm)

    plsc.subcore_barrier()

    pltpu.sync_copy(sw_sh.at[pl.ds(wid * SPW, SPW)], w_own_v)
    pltpu.sync_copy(w_own_v, wsort_hbm.at[pl.ds(wid * SPW, SPW)])
    pltpu.sync_copy(stok_sh.at[pl.ds(wid * SPW, SPW)], slot_v)

    def cl(i, c):
        sl = pl.ds(i * 16, 16)
        slot_v[sl] = slot_v[sl] & (T - 1)
        return c

    lax.fori_loop(0, SPW // 16, cl, 0)

    rows = (rows0_v, rows1_v)
    sems = (sem0, sem1)
    nch = SPW // GCH

    def issue(ci):
        return pltpu.async_copy(x_hbm.at[slot_v.at[pl.ds(ci * GCH, GCH)]],
                                rows[ci % 2], sems[ci % 2])

    inflight = issue(0)
    for ci in range(nch):
        nxt = issue(ci + 1) if ci + 1 < nch else None
        inflight.wait()
        pltpu.sync_copy(rows[ci % 2],
                        xs_hbm.at[pl.ds(wid * SPW + ci * GCH, GCH)])
        inflight = nxt


@functools.cache
def _dispatch():
    return pl.kernel(
        _dispatch_body,
        out_type=[
            jax.ShapeDtypeStruct((S,), jnp.float32),
            jax.ShapeDtypeStruct((T,), jnp.int32),
            jax.ShapeDtypeStruct((T,), jnp.int32),
            jax.ShapeDtypeStruct((48,), jnp.int32),
            jax.ShapeDtypeStruct((S, D), jnp.float32),
        ],
        mesh=_sc_mesh(),
        scratch_types=[
            pltpu.VMEM((16,), jnp.int32),
            pltpu.VMEM((16,), jnp.int32),
            pltpu.VMEM((16,), jnp.int32),
            pltpu.VMEM((TPC,), jnp.int32),
            pltpu.VMEM((TPC,), jnp.int32),
            pltpu.VMEM((TPC,), jnp.int32),
            pltpu.VMEM((TPC,), jnp.int32),
            pltpu.VMEM((TPC,), jnp.float32),
            pltpu.VMEM((TPC,), jnp.float32),
            pltpu.VMEM((TPC,), jnp.int32),
            pltpu.VMEM((TPC,), jnp.int32),
            pltpu.VMEM((TPC,), jnp.int32),
            pltpu.VMEM((SPW,), jnp.int32),
            pltpu.VMEM((SPW,), jnp.float32),
            pltpu.VMEM((48,), jnp.int32),
            pltpu.VMEM_SHARED((S,), jnp.int32),
            pltpu.VMEM_SHARED((S,), jnp.float32),
            pltpu.VMEM((GCH, D), jnp.float32),
            pltpu.VMEM((GCH, D), jnp.float32),
            pltpu.SemaphoreType.DMA,
            pltpu.SemaphoreType.DMA,
        ],
        compiler_params=pltpu.CompilerParams(needs_layout_passes=False),
    )


def _gemm_body(be_ref, xs_ref, w1_ref, w3_ref, w2_ref, ws_ref, out_ref):
    g_id = pl.program_id(0)

    @pl.when(g_id < be_ref[GMAX])
    def _():
        xb = xs_ref[...]
        h1 = jnp.dot(xb, w1_ref[0], preferred_element_type=jnp.float32)
        h3 = jnp.dot(xb, w3_ref[0], preferred_element_type=jnp.float32)
        g = h1 * jax.nn.sigmoid(h1) * h3
        o = jnp.dot(g, w2_ref[0], preferred_element_type=jnp.float32)
        wrow = ws_ref[...].reshape(1, B)
        eye = (lax.broadcasted_iota(jnp.int32, (B, B), 0)
               == lax.broadcasted_iota(jnp.int32, (B, B), 1))
        wcol = jnp.sum(jnp.where(eye, wrow, jnp.zeros((B, B), jnp.float32)),
                       axis=1, keepdims=True)
        out_ref[...] = o * wcol


def _gemm(be, xs, w1, w3, w2, ws):
    grid_spec = pltpu.PrefetchScalarGridSpec(
        num_scalar_prefetch=1,
        grid=(GMAX,),
        in_specs=[
            pl.BlockSpec((B, D), lambda g, be_ref: (g, 0)),
            pl.BlockSpec((1, D, F), lambda g, be_ref: (be_ref[g], 0, 0)),
            pl.BlockSpec((1, D, F), lambda g, be_ref: (be_ref[g], 0, 0)),
            pl.BlockSpec((1, F, D), lambda g, be_ref: (be_ref[g], 0, 0)),
            pl.BlockSpec((B,), lambda g, be_ref: (g,)),
        ],
        out_specs=pl.BlockSpec((B, D), lambda g, be_ref: (g, 0)),
    )
    return pl.pallas_call(
        _gemm_body,
        grid_spec=grid_spec,
        out_shape=jax.ShapeDtypeStruct((S, D), jnp.float32),
        compiler_params=pltpu.CompilerParams(
            dimension_semantics=("arbitrary",),
        ),
    )(be, xs, w1, w3, w2, ws)


def _combine_body(outs_hbm, pos0_hbm, pos1_hbm, y_hbm,
                  p0_v, p1_v, a0_v, a1_v, b0_v, b1_v,
                  sa0, sa1, sb0, sb1):
    wid = lax.axis_index("s") * NC + lax.axis_index("c")
    pltpu.sync_copy(pos0_hbm.at[pl.ds(wid * TPW, TPW)], p0_v)
    pltpu.sync_copy(pos1_hbm.at[pl.ds(wid * TPW, TPW)], p1_v)

    sets = ((a0_v, a1_v, sa0, sa1), (b0_v, b1_v, sb0, sb1))
    nch = TPW // CCH

    def issue(ci):
        r0, r1, s0, s1 = sets[ci % 2]
        return (
            pltpu.async_copy(outs_hbm.at[p0_v.at[pl.ds(ci * CCH, CCH)]],
                             r0, s0),
            pltpu.async_copy(outs_hbm.at[p1_v.at[pl.ds(ci * CCH, CCH)]],
                             r1, s1),
        )

    inflight = issue(0)
    for ci in range(nch):
        nxt = issue(ci + 1) if ci + 1 < nch else None
        inflight[0].wait()
        inflight[1].wait()
        r0, r1, _, _ = sets[ci % 2]

        def arow(rr, c2, r0=r0, r1=r1):
            for cc in range(D // 16):
                sl = pl.ds(cc * 16, 16)
                r0[rr, sl] = r0[rr, sl] + r1[rr, sl]
            return c2

        lax.fori_loop(0, CCH, arow, 0)
        pltpu.sync_copy(r0, y_hbm.at[pl.ds(wid * TPW + ci * CCH, CCH)])
        inflight = nxt


@functools.cache
def _combine():
    return pl.kernel(
        _combine_body,
        out_type=jax.ShapeDtypeStruct((T, D), jnp.float32),
        mesh=_sc_mesh(),
        scratch_types=[
            pltpu.VMEM((TPW,), jnp.int32),
            pltpu.VMEM((TPW,), jnp.int32),
            pltpu.VMEM((CCH, D), jnp.float32),
            pltpu.VMEM((CCH, D), jnp.float32),
            pltpu.VMEM((CCH, D), jnp.float32),
            pltpu.VMEM((CCH, D), jnp.float32),
            pltpu.SemaphoreType.DMA,
            pltpu.SemaphoreType.DMA,
            pltpu.SemaphoreType.DMA,
            pltpu.SemaphoreType.DMA,
        ],
        compiler_params=pltpu.CompilerParams(needs_layout_passes=False),
    )


def kernel(hidden_states, W_gate, W1, W3, W2):
    x = hidden_states
    idx0, idx1, rank0, rank1, w0, w1, base16, nb16, cumnb16 = _router(x, W_gate)
    wsort, pos0, pos1, be, xs = _dispatch()(
        base16, nb16, cumnb16, idx0, idx1, rank0, rank1, w0, w1, x)
    outs = _gemm(be, xs, W1, W3, W2, wsort)
    y = _combine()(outs, pos0, pos1)
    return y


# speedup vs baseline: 1.0303x; 1.0045x over previous
import functools

import jax
import jax.numpy as jnp
from jax import lax
from jax.experimental import pallas as pl
from jax.experimental.pallas import tpu as pltpu
from jax.experimental.pallas import tpu_sc as plsc

T = 4096
D = 1024
F = 1408
E = 8
BT = 256
N_TB = T // BT

B = 512
GMAX = (T * 2) // B + E
S = GMAX * B

NC = 2
NS = 16
NW = NC * NS
TPW = T // NW
SPW = S // NW
GCH = 48
CCH = 16


def _col2row(col, n):
    eye = (lax.broadcasted_iota(jnp.int32, (n, n), 0)
           == lax.broadcasted_iota(jnp.int32, (n, n), 1))
    return jnp.sum(jnp.where(eye, col, jnp.zeros_like(col)), axis=0)


def _router_body(x_ref, wg_ref, idx0_ref, idx1_ref, rank0_ref, rank1_ref,
                 w0_ref, w1_ref, base_ref, nb_ref, cumnb_ref, carry_ref):
    tb = pl.program_id(0)
    xb = x_ref[...]
    logits = jnp.dot(xb, wg_ref[...], preferred_element_type=jnp.float32)
    m = jnp.max(logits, axis=1, keepdims=True)
    ex = jnp.exp(logits - m)
    p = ex / jnp.sum(ex, axis=1, keepdims=True)
    iota = lax.broadcasted_iota(jnp.int32, (BT, E), 1)
    m1 = jnp.max(p, axis=1, keepdims=True)
    i1 = jnp.min(jnp.where(p >= m1, iota, E), axis=1, keepdims=True)
    sel1 = iota == i1
    p2 = jnp.where(sel1, -jnp.inf, p)
    m2 = jnp.max(p2, axis=1, keepdims=True)
    i2 = jnp.min(jnp.where(p2 >= m2, iota, E), axis=1, keepdims=True)
    sel2 = iota == i2
    s = m1 + m2

    @pl.when(tb == 0)
    def _():
        carry_ref[...] = jnp.zeros((1, E), jnp.float32)

    carry = carry_ref[...]
    oh = sel1.astype(jnp.float32) + sel2.astype(jnp.float32)
    tri = (lax.broadcasted_iota(jnp.int32, (BT, BT), 0)
           > lax.broadcasted_iota(jnp.int32, (BT, BT), 1)).astype(jnp.float32)
    cum_excl = carry + jnp.dot(tri, oh, preferred_element_type=jnp.float32)
    zero = jnp.zeros((BT, E), jnp.float32)
    rank0 = jnp.sum(jnp.where(sel1, cum_excl, zero), axis=1, keepdims=True)
    rank1 = jnp.sum(jnp.where(sel2, cum_excl, zero), axis=1, keepdims=True)
    new_carry = carry + jnp.sum(oh, axis=0, keepdims=True)
    carry_ref[...] = new_carry

    idx0_ref[...] = _col2row(i1, BT)
    idx1_ref[...] = _col2row(i2, BT)
    rank0_ref[...] = _col2row(rank0.astype(jnp.int32), BT)
    rank1_ref[...] = _col2row(rank1.astype(jnp.int32), BT)
    w0_ref[...] = _col2row(m1 / s, BT)
    w1_ref[...] = _col2row(m2 / s, BT)
    padded = jnp.ceil(new_carry / B) * B
    nb = padded / B
    strict = (lax.broadcasted_iota(jnp.int32, (E, E), 0)
              < lax.broadcasted_iota(jnp.int32, (E, E), 1)).astype(jnp.float32)
    base_excl = jnp.dot(padded, strict, preferred_element_type=jnp.float32)
    cumnb_excl = jnp.dot(nb, strict, preferred_element_type=jnp.float32)
    zpad = jnp.zeros((1, E), jnp.int32)

    def _pack16(row):
        return jnp.concatenate([row.astype(jnp.int32), zpad], axis=1).reshape(16)

    base_ref[...] = _pack16(base_excl)
    nb_ref[...] = _pack16(nb)
    cumnb_ref[...] = _pack16(cumnb_excl)


def _router(x, wg):
    return pl.pallas_call(
        _router_body,
        grid=(N_TB,),
        in_specs=[
            pl.BlockSpec((BT, D), lambda t: (t, 0)),
            pl.BlockSpec((D, E), lambda t: (0, 0)),
        ],
        out_specs=[
            pl.BlockSpec((BT,), lambda t: (t,)),
            pl.BlockSpec((BT,), lambda t: (t,)),
            pl.BlockSpec((BT,), lambda t: (t,)),
            pl.BlockSpec((BT,), lambda t: (t,)),
            pl.BlockSpec((BT,), lambda t: (t,)),
            pl.BlockSpec((BT,), lambda t: (t,)),
            pl.BlockSpec((16,), lambda t: (0,)),
            pl.BlockSpec((16,), lambda t: (0,)),
            pl.BlockSpec((16,), lambda t: (0,)),
        ],
        out_shape=[
            jax.ShapeDtypeStruct((T,), jnp.int32),
            jax.ShapeDtypeStruct((T,), jnp.int32),
            jax.ShapeDtypeStruct((T,), jnp.int32),
            jax.ShapeDtypeStruct((T,), jnp.int32),
            jax.ShapeDtypeStruct((T,), jnp.float32),
            jax.ShapeDtypeStruct((T,), jnp.float32),
            jax.ShapeDtypeStruct((16,), jnp.int32),
            jax.ShapeDtypeStruct((16,), jnp.int32),
            jax.ShapeDtypeStruct((16,), jnp.int32),
        ],
        scratch_shapes=[pltpu.VMEM((1, E), jnp.float32)],
    )(x, wg)


@functools.cache
def _sc_mesh():
    return plsc.VectorSubcoreMesh(core_axis_name="c", subcore_axis_name="s",
                                  num_cores=NC, num_subcores=NS)


TPC = T // NS


def _dispatch_body(base_hbm, nb_hbm, cumnb_hbm, idx0_hbm, idx1_hbm,
                   rank0_hbm, rank1_hbm, w0_hbm, w1_hbm, x_hbm,
                   wsort_hbm, pos0_hbm, pos1_hbm, be_hbm, xs_hbm,
                   base_v, nb_v, cumnb_v, idx0_v, idx1_v, rank0_v, rank1_v,
                   w0_v, w1_v, p0_v, p1_v, tok_v, slot_v, w_own_v, bev_v,
                   stok_sh, sw_sh, rows0_v, rows1_v, sem0, sem1):
    cid = lax.axis_index("c")
    sid = lax.axis_index("s")
    wid = sid * NC + cid
    tbase = sid * TPC

    cps = [pltpu.async_copy(src, dst, sem0) for src, dst in (
        (base_hbm, base_v), (nb_hbm, nb_v), (cumnb_hbm, cumnb_v),
        (idx0_hbm.at[pl.ds(tbase, TPC)], idx0_v),
        (idx1_hbm.at[pl.ds(tbase, TPC)], idx1_v),
        (rank0_hbm.at[pl.ds(tbase, TPC)], rank0_v),
        (rank1_hbm.at[pl.ds(tbase, TPC)], rank1_v),
        (w0_hbm.at[pl.ds(tbase, TPC)], w0_v),
        (w1_hbm.at[pl.ds(tbase, TPC)], w1_v))]
    for cp in cps:
        cp.wait()

    lane = lax.broadcasted_iota(jnp.int32, (16,), 0)

    def pbody(i, c):
        sl = pl.ds(i * 16, 16)
        tok_v[sl] = tbase + i * 16 + lane
        p0_v[sl] = plsc.load_gather(base_v, [idx0_v[sl]]) + rank0_v[sl]
        p1_v[sl] = plsc.load_gather(base_v, [idx1_v[sl]]) + rank1_v[sl]
        return c

    lax.fori_loop(0, TPC // 16, pbody, 0)

    pltpu.sync_copy(tok_v, stok_sh.at[p0_v])
    pltpu.sync_copy(tok_v, stok_sh.at[p1_v])
    pltpu.sync_copy(w0_v, sw_sh.at[p0_v])
    pltpu.sync_copy(w1_v, sw_sh.at[p1_v])

    @pl.when(cid == 0)
    def _():
        pltpu.sync_copy(p0_v, pos0_hbm.at[pl.ds(tbase, TPC)])
        pltpu.sync_copy(p1_v, pos1_hbm.at[pl.ds(tbase, TPC)])

    @pl.when(wid == 0)
    def _():
        nb = nb_v[...]
        cumnb_excl = cumnb_v[...]

        def ib(i, c):
            bev_v[pl.ds(i * 16, 16)] = jnp.full((16,), E - 1, jnp.int32)
            return c

        lax.fori_loop(0, 3, ib, 0)
        for j in range(16):
            plsc.store_scatter(bev_v, [cumnb_excl + j], lane, mask=j < nb)
        gu = cumnb_excl + nb
        plsc.store_scatter(bev_v, [jnp.where(lane == 7, GMAX, 47)], gu,
                           mask=lane == 7)
        pltpu.sync_copy(bev_v, be_hbm)

    plsc.subcore_barrier()

    pltpu.sync_copy(sw_sh.at[pl.ds(wid * SPW, SPW)], w_own_v)
    pltpu.sync_copy(w_own_v, wsort_hbm.at[pl.ds(wid * SPW, SPW)])
    pltpu.sync_copy(stok_sh.at[pl.ds(wid * SPW, SPW)], slot_v)

    def cl(i, c):
        sl = pl.ds(i * 16, 16)
        slot_v[sl] = slot_v[sl] & (T - 1)
        return c

    lax.fori_loop(0, SPW // 16, cl, 0)

    rows = (rows0_v, rows1_v)
    sems = (sem0, sem1)
    nch = SPW // GCH

    def issue(ci):
        return pltpu.async_copy(x_hbm.at[slot_v.at[pl.ds(ci * GCH, GCH)]],
                                rows[ci % 2], sems[ci % 2])

    inflight = issue(0)
    for ci in range(nch):
        nxt = issue(ci + 1) if ci + 1 < nch else None
        inflight.wait()
        pltpu.sync_copy(rows[ci % 2],
                        xs_hbm.at[pl.ds(wid * SPW + ci * GCH, GCH)])
        inflight = nxt


@functools.cache
def _dispatch():
    return pl.kernel(
        _dispatch_body,
        out_type=[
            jax.ShapeDtypeStruct((S,), jnp.float32),
            jax.ShapeDtypeStruct((T,), jnp.int32),
            jax.ShapeDtypeStruct((T,), jnp.int32),
            jax.ShapeDtypeStruct((48,), jnp.int32),
            jax.ShapeDtypeStruct((S, D), jnp.float32),
        ],
        mesh=_sc_mesh(),
        scratch_types=[
            pltpu.VMEM((16,), jnp.int32),
            pltpu.VMEM((16,), jnp.int32),
            pltpu.VMEM((16,), jnp.int32),
            pltpu.VMEM((TPC,), jnp.int32),
            pltpu.VMEM((TPC,), jnp.int32),
            pltpu.VMEM((TPC,), jnp.int32),
            pltpu.VMEM((TPC,), jnp.int32),
            pltpu.VMEM((TPC,), jnp.float32),
            pltpu.VMEM((TPC,), jnp.float32),
            pltpu.VMEM((TPC,), jnp.int32),
            pltpu.VMEM((TPC,), jnp.int32),
            pltpu.VMEM((TPC,), jnp.int32),
            pltpu.VMEM((SPW,), jnp.int32),
            pltpu.VMEM((SPW,), jnp.float32),
            pltpu.VMEM((48,), jnp.int32),
            pltpu.VMEM_SHARED((S,), jnp.int32),
            pltpu.VMEM_SHARED((S,), jnp.float32),
            pltpu.VMEM((GCH, D), jnp.float32),
            pltpu.VMEM((GCH, D), jnp.float32),
            pltpu.SemaphoreType.DMA,
            pltpu.SemaphoreType.DMA,
        ],
        compiler_params=pltpu.CompilerParams(needs_layout_passes=False),
    )


def _gemm_body(be_ref, xs_ref, w1_ref, w3_ref, w2_ref, ws_ref, out_ref):
    g_id = pl.program_id(0)

    @pl.when(g_id < be_ref[GMAX])
    def _():
        xb = xs_ref[...]
        h1 = jnp.dot(xb, w1_ref[0], preferred_element_type=jnp.float32)
        h3 = jnp.dot(xb, w3_ref[0], preferred_element_type=jnp.float32)
        g = h1 * jax.nn.sigmoid(h1) * h3
        o = jnp.dot(g, w2_ref[0], preferred_element_type=jnp.float32)
        wrow = ws_ref[...].reshape(1, B)
        eye = (lax.broadcasted_iota(jnp.int32, (B, B), 0)
               == lax.broadcasted_iota(jnp.int32, (B, B), 1))
        wcol = jnp.sum(jnp.where(eye, wrow, jnp.zeros((B, B), jnp.float32)),
                       axis=1, keepdims=True)
        out_ref[...] = o * wcol


def _gemm(be, xs, w1, w3, w2, ws):
    grid_spec = pltpu.PrefetchScalarGridSpec(
        num_scalar_prefetch=1,
        grid=(GMAX,),
        in_specs=[
            pl.BlockSpec((B, D), lambda g, be_ref: (g, 0)),
            pl.BlockSpec((1, D, F), lambda g, be_ref: (be_ref[g], 0, 0)),
            pl.BlockSpec((1, D, F), lambda g, be_ref: (be_ref[g], 0, 0)),
            pl.BlockSpec((1, F, D), lambda g, be_ref: (be_ref[g], 0, 0)),
            pl.BlockSpec((B,), lambda g, be_ref: (g,)),
        ],
        out_specs=pl.BlockSpec((B, D), lambda g, be_ref: (g, 0)),
    )
    return pl.pallas_call(
        _gemm_body,
        grid_spec=grid_spec,
        out_shape=jax.ShapeDtypeStruct((S, D), jnp.float32),
        compiler_params=pltpu.CompilerParams(
            dimension_semantics=("arbitrary",),
        ),
    )(be, xs, w1, w3, w2, ws)


def _combine_body(outs_hbm, pos0_hbm, pos1_hbm, y_hbm,
                  p0_v, p1_v, a0_v, a1_v, b0_v, b1_v,
                  sa0, sa1, sb0, sb1):
    wid = lax.axis_index("s") * NC + lax.axis_index("c")
    pltpu.sync_copy(pos0_hbm.at[pl.ds(wid * TPW, TPW)], p0_v)
    pltpu.sync_copy(pos1_hbm.at[pl.ds(wid * TPW, TPW)], p1_v)

    sets = ((a0_v, a1_v, sa0, sa1), (b0_v, b1_v, sb0, sb1))
    nch = TPW // CCH

    def issue(ci):
        r0, r1, s0, s1 = sets[ci % 2]
        return (
            pltpu.async_copy(outs_hbm.at[p0_v.at[pl.ds(ci * CCH, CCH)]],
                             r0, s0),
            pltpu.async_copy(outs_hbm.at[p1_v.at[pl.ds(ci * CCH, CCH)]],
                             r1, s1),
        )

    inflight = issue(0)
    for ci in range(nch):
        nxt = issue(ci + 1) if ci + 1 < nch else None
        inflight[0].wait()
        inflight[1].wait()
        r0, r1, _, _ = sets[ci % 2]

        def arow(rr, c2, r0=r0, r1=r1):
            for cc in range(D // 16):
                sl = pl.ds(cc * 16, 16)
                r0[rr, sl] = r0[rr, sl] + r1[rr, sl]
            return c2

        lax.fori_loop(0, CCH, arow, 0)
        pltpu.sync_copy(r0, y_hbm.at[pl.ds(wid * TPW + ci * CCH, CCH)])
        inflight = nxt


@functools.cache
def _combine():
    return pl.kernel(
        _combine_body,
        out_type=jax.ShapeDtypeStruct((T, D), jnp.float32),
        mesh=_sc_mesh(),
        scratch_types=[
            pltpu.VMEM((TPW,), jnp.int32),
            pltpu.VMEM((TPW,), jnp.int32),
            pltpu.VMEM((CCH, D), jnp.float32),
            pltpu.VMEM((CCH, D), jnp.float32),
            pltpu.VMEM((CCH, D), jnp.float32),
            pltpu.VMEM((CCH, D), jnp.float32),
            pltpu.SemaphoreType.DMA,
            pltpu.SemaphoreType.DMA,
            pltpu.SemaphoreType.DMA,
            pltpu.SemaphoreType.DMA,
        ],
        compiler_params=pltpu.CompilerParams(needs_layout_passes=False),
    )


def kernel(hidden_states, W_gate, W1, W3, W2):
    x = hidden_states
    idx0, idx1, rank0, rank1, w0, w1, base16, nb16, cumnb16 = _router(x, W_gate)
    wsort, pos0, pos1, be, xs = _dispatch()(
        base16, nb16, cumnb16, idx0, idx1, rank0, rank1, w0, w1, x)
    outs = _gemm(be, xs, W1, W3, W2, wsort)
    y = _combine()(outs, pos0, pos1)
    return y
